# baseline (speedup 1.0000x reference)
import functools

import jax
import jax.numpy as jnp
from jax import lax
from jax.experimental import pallas as pl
from jax.experimental.pallas import tpu as pltpu
from jax.experimental.pallas import tpu_sc as plsc

N = 10000
E = 320000
D = 128
H = 128
C = 16
G = 128

NP = 10240
BLK = 1024
NBLK = NP // BLK
NC = 2
NS = 16
NWORK = NC * NS
EPW = E // NWORK
K = 80
NB = EPW // K
NB3 = (NB - 2) // 3
ROWS_PW = NP // NS


def _front_body(x_ref, w_ref, aw_ref, z_ref, sp_ref, stats_ref, ssum, ssq):
    i = pl.program_id(0)

    @pl.when(i == 0)
    def _():
        ssum[0] = 0.0
        ssq[0] = 0.0

    xb = x_ref[...]
    ssum[0] += jnp.sum(xb)
    ssq[0] += jnp.sum(xb * xb)

    w = w_ref[...]
    zb = jnp.dot(xb, w.T, preferred_element_type=jnp.float32)
    z_ref[...] = zb

    aw = aw_ref[...]
    a1 = aw[0, :H]
    a2 = aw[0, H:]
    rid = lax.broadcasted_iota(jnp.int32, (16, H), 0)
    amat = jnp.where(rid == 0, a1[None, :],
                     jnp.where(rid == 1, a2[None, :], 0.0))
    sp_ref[...] = lax.dot_general(amat, zb, (((1,), (1,)), ((), ())),
                                  preferred_element_type=jnp.float32)

    @pl.when(i == pl.num_programs(0) - 1)
    def _():
        cnt = float(N * D)
        mu = ssum[0] / cnt
        var = (ssq[0] - cnt * mu * mu) / (cnt - 1.0)
        rstd = lax.rsqrt(var)
        wsum = jnp.sum(w, axis=1)
        k12 = jnp.sum(wsum * (a1 + a2))
        c12 = mu * rstd * k12
        lane = lax.broadcasted_iota(jnp.int32, (1, H), 1)
        row1 = jnp.where(lane == 0, mu,
                         jnp.where(lane == 1, rstd,
                                   jnp.where(lane == 2, c12, 0.0)))
        stats_ref[0:1, :] = (mu * rstd) * wsum[None, :]
        stats_ref[1:2, :] = row1
        stats_ref[2:8, :] = jnp.zeros((6, H), jnp.float32)


def _front(x_p, W_fc, a_w):
    return pl.pallas_call(
        _front_body,
        grid=(NBLK,),
        in_specs=[
            pl.BlockSpec((BLK, D), lambda i: (i, 0)),
            pl.BlockSpec((H, D), lambda i: (0, 0)),
            pl.BlockSpec((1, 2 * H), lambda i: (0, 0)),
        ],
        out_specs=[
            pl.BlockSpec((BLK, H), lambda i: (i, 0)),
            pl.BlockSpec((16, BLK), lambda i: (0, i)),
            pl.BlockSpec((8, H), lambda i: (0, 0)),
        ],
        out_shape=[
            jax.ShapeDtypeStruct((NP, H), jnp.float32),
            jax.ShapeDtypeStruct((16, NP), jnp.float32),
            jax.ShapeDtypeStruct((8, H), jnp.float32),
        ],
        scratch_shapes=[pltpu.SMEM((1,), jnp.float32),
                        pltpu.SMEM((1,), jnp.float32)],
    )(x_p, W_fc, a_w)


def _sc_h_body(src_hbm, dst_hbm, sp_hbm, stats_hbm, out_h_hbm, out_hs_hbm,
               s1_v, s2_v, st_v, src_v, dst_v, h_v, hsum_v):
    c = lax.axis_index("c")
    s = lax.axis_index("s")
    wid = s * NC + c

    pltpu.sync_copy(sp_hbm.at[0], s1_v)
    pltpu.sync_copy(sp_hbm.at[1], s2_v)
    pltpu.sync_copy(stats_hbm.at[1, pl.ds(0, 16)], st_v)
    stv = st_v[...]
    rstd = stv[1]
    c12 = stv[2]

    def zh(t, carry):
        hsum_v[pl.ds(t * 16, 16)] = jnp.zeros((16,), jnp.float32)
        return carry

    lax.fori_loop(0, NP // 16, zh, 0)

    base0 = wid * EPW
    pltpu.sync_copy(src_hbm.at[pl.ds(base0, EPW)], src_v)
    pltpu.sync_copy(dst_hbm.at[pl.ds(base0, EPW)], dst_v)

    def grp(j, carry):
        s16 = src_v[pl.ds(j * 16, 16)]
        d16 = dst_v[pl.ds(j * 16, 16)]
        g = plsc.load_gather(s1_v, [s16]) + plsc.load_gather(s2_v, [d16])
        lg = g * rstd - c12
        lg = jnp.where(lg >= 0.0, lg, 0.05 * lg)
        h16 = jnp.exp(lg)
        h_v[pl.ds(j * 16, 16)] = h16
        plsc.addupdate_scatter(hsum_v, [s16], h16)
        return carry

    lax.fori_loop(0, EPW // 16, grp, 0)
    pltpu.sync_copy(h_v, out_h_hbm.at[pl.ds(base0, EPW)])
    pltpu.sync_copy(hsum_v, out_hs_hbm.at[wid])


def _sc_h(src, dst, s_pair, stats):
    mesh = plsc.VectorSubcoreMesh(core_axis_name="c", subcore_axis_name="s")
    f = pl.kernel(
        _sc_h_body,
        out_type=[
            jax.ShapeDtypeStruct((E,), jnp.float32),
            jax.ShapeDtypeStruct((NWORK, NP), jnp.float32),
        ],
        mesh=mesh,
        scratch_types=[
            pltpu.VMEM((NP,), jnp.float32),
            pltpu.VMEM((NP,), jnp.float32),
            pltpu.VMEM((16,), jnp.float32),
            pltpu.VMEM((EPW,), jnp.int32),
            pltpu.VMEM((EPW,), jnp.int32),
            pltpu.VMEM((EPW,), jnp.float32),
            pltpu.VMEM((NP,), jnp.float32),
        ],
        compiler_params=pltpu.CompilerParams(needs_layout_passes=False),
    )
    return f(src, dst, s_pair, stats)


NBUF = 4
NB4 = (NB - 1) // NBUF


def _sc_scatter_body(src_hbm, dst_hbm, h_hbm, z_hbm, out_u_hbm,
                     s0_v, s1_v, s2_v, s3_v, d0_v, d1_v, d2_v, d3_v,
                     h0_v, h1_v, h2_v, h3_v, r0_v, r1_v, r2_v, r3_v, u_sh,
                     i0, i1, i2, i3, g0, g1, g2, g3, t0, t1, t2, t3):
    c = lax.axis_index("c")
    s = lax.axis_index("s")
    wid = s * NC + c
    base0 = wid * EPW

    src_bufs = (s0_v, s1_v, s2_v, s3_v)
    dst_bufs = (d0_v, d1_v, d2_v, d3_v)
    h_bufs = (h0_v, h1_v, h2_v, h3_v)
    row_bufs = (r0_v, r1_v, r2_v, r3_v)
    isems = (i0, i1, i2, i3)
    gsems = (g0, g1, g2, g3)
    ssems = (t0, t1, t2, t3)

    def zrow(e, carry):
        for q in range(H // 16):
            r0_v[e, pl.ds(q * 16, 16)] = jnp.zeros((16,), jnp.float32)
        return carry

    lax.fori_loop(0, K, zrow, 0)
    row0 = s * ROWS_PW
    for j in range(ROWS_PW // K):
        pltpu.sync_copy(r0_v, u_sh.at[pl.ds(row0 + j * K, K)])
    plsc.subcore_barrier()

    def stage_idx(b, par):
        pltpu.async_copy(src_hbm.at[pl.ds(base0 + b * K, K)], src_bufs[par],
                         isems[par])
        pltpu.async_copy(dst_hbm.at[pl.ds(base0 + b * K, K)], dst_bufs[par],
                         isems[par])
        pltpu.async_copy(h_hbm.at[pl.ds(base0 + b * K, K)], h_bufs[par],
                         isems[par])

    def wait_idx(par):
        pltpu.make_async_copy(src_hbm.at[pl.ds(base0, K)], src_bufs[par],
                              isems[par]).wait()
        pltpu.make_async_copy(dst_hbm.at[pl.ds(base0, K)], dst_bufs[par],
                              isems[par]).wait()
        pltpu.make_async_copy(h_hbm.at[pl.ds(base0, K)], h_bufs[par],
                              isems[par]).wait()

    def start_gather(par):
        pltpu.async_copy(z_hbm.at[dst_bufs[par]], row_bufs[par], gsems[par])

    def wait_gather(par):
        pltpu.make_async_copy(z_hbm.at[dst_bufs[par]], row_bufs[par],
                              gsems[par]).wait()

    def scale(par):
        rows = row_bufs[par]
        hb = h_bufs[par]

        def erow(e, carry):
            ev = jnp.full((16,), 0, jnp.int32) + e
            hs = plsc.load_gather(hb, [ev])
            for q in range(H // 16):
                rows[e, pl.ds(q * 16, 16)] = rows[e, pl.ds(q * 16, 16)] * hs
            return carry

        lax.fori_loop(0, K, erow, 0)

    def start_scatter(par):
        pltpu.async_copy(row_bufs[par], u_sh.at[src_bufs[par]], ssems[par],
                         add=True)

    def wait_scatter(par):
        pltpu.make_async_copy(row_bufs[par], u_sh.at[src_bufs[par]],
                              ssems[par]).wait()

    stage_idx(0, 0)
    stage_idx(1, 1)
    wait_idx(0)
    start_gather(0)

    def step(b, par):
        wait_gather(par)
        scale(par)
        start_scatter(par)
        nb2 = b + 2
        pn2 = (par + 2) % NBUF

        @pl.when(nb2 < NB)
        def _():
            @pl.when(b >= 2)
            def _():
                wait_scatter(pn2)

            stage_idx(nb2, pn2)

        pn1 = (par + 1) % NBUF

        @pl.when(b + 1 < NB)
        def _():
            wait_idx(pn1)
            start_gather(pn1)

    def quad(q, carry):
        b = q * NBUF
        step(b, 0)
        step(b + 1, 1)
        step(b + 2, 2)
        step(b + 3, 3)
        return carry

    lax.fori_loop(0, NB4, quad, 0)
    step(NB - 1, (NB - 1) % NBUF)

    for b in (NB - 4, NB - 3, NB - 2, NB - 1):
        wait_scatter(b % NBUF)

    plsc.subcore_barrier()
    pltpu.sync_copy(u_sh.at[pl.ds(row0, ROWS_PW)],
                    out_u_hbm.at[c, pl.ds(row0, ROWS_PW)])


def _sc_scatter(src, dst, h_all, zraw):
    mesh = plsc.VectorSubcoreMesh(core_axis_name="c", subcore_axis_name="s")
    f = pl.kernel(
        _sc_scatter_body,
        out_type=jax.ShapeDtypeStruct((NC, NP, H), jnp.float32),
        mesh=mesh,
        scratch_types=(
            [pltpu.VMEM((K,), jnp.int32)] * NBUF
            + [pltpu.VMEM((K,), jnp.int32)] * NBUF
            + [pltpu.VMEM((K,), jnp.float32)] * NBUF
            + [pltpu.VMEM((K, H), jnp.float32)] * NBUF
            + [pltpu.VMEM_SHARED((NP, H), jnp.float32)]
            + [pltpu.SemaphoreType.DMA] * (3 * NBUF)
        ),
        compiler_params=pltpu.CompilerParams(needs_layout_passes=False),
    )
    return f(src, dst, h_all, zraw)


def _post_body(p_ref, h_ref, idx_ref, stats_ref, gamma_ref, beta_ref,
               fc1w_ref, fc1b_ref, fc2w_ref, fc2b_ref, out_ref, acc_ref):
    i = pl.program_id(0)

    @pl.when(i == 0)
    def _():
        acc_ref[...] = jnp.zeros_like(acc_ref)

    u = p_ref[0] + p_ref[1]
    hs = jnp.sum(h_ref[...], axis=0)[:, None]
    muwr = stats_ref[0:1, :]
    rstd = stats_ref[1, 1]
    hs_safe = jnp.where(hs > 0.0, hs, 1.0)
    xh = jnp.maximum((u * rstd - hs * muwr) / hs_safe, 0.0)

    idxb = idx_ref[0, 0, :]
    oh = (idxb[:, None] == lax.broadcasted_iota(jnp.int32, (1, G), 1)
          ).astype(jnp.float32)
    acc_ref[...] += lax.dot_general(oh, xh, (((0,), (0,)), ((), ())),
                                    preferred_element_type=jnp.float32)

    @pl.when(i == pl.num_programs(0) - 1)
    def _():
        pooled = acc_ref[...]
        mean = jnp.mean(pooled, axis=0, keepdims=True)
        var = jnp.mean((pooled - mean) ** 2, axis=0, keepdims=True)
        xb = (pooled - mean) * lax.rsqrt(var + 1e-5)
        xb = xb * gamma_ref[...] + beta_ref[...]
        y = jnp.maximum(
            jnp.dot(xb, fc1w_ref[...].T, preferred_element_type=jnp.float32)
            + fc1b_ref[...], 0.0)
        y = (jnp.dot(y, fc2w_ref[...].T, preferred_element_type=jnp.float32)
             + fc2b_ref[...])
        m = jnp.max(y, axis=1, keepdims=True)
        ly = y - m
        lse = jnp.log(jnp.sum(jnp.exp(ly), axis=1, keepdims=True))
        out_ref[...] = ly - lse


def _post(partials, hsums, idx3, stats, bn_gamma, bn_beta, fc1_w, fc1_b,
          fc2_w, fc2_b):
    return pl.pallas_call(
        _post_body,
        grid=(NBLK,),
        in_specs=[
            pl.BlockSpec((NC, BLK, H), lambda i: (0, i, 0)),
            pl.BlockSpec((NWORK, BLK), lambda i: (0, i)),
            pl.BlockSpec((1, 1, BLK), lambda i: (i, 0, 0)),
            pl.BlockSpec((8, H), lambda i: (0, 0)),
            pl.BlockSpec((1, H), lambda i: (0, 0)),
            pl.BlockSpec((1, H), lambda i: (0, 0)),
            pl.BlockSpec((H, H), lambda i: (0, 0)),
            pl.BlockSpec((1, H), lambda i: (0, 0)),
            pl.BlockSpec((C, H), lambda i: (0, 0)),
            pl.BlockSpec((1, C), lambda i: (0, 0)),
        ],
        out_specs=pl.BlockSpec((G, C), lambda i: (0, 0)),
        out_shape=jax.ShapeDtypeStruct((G, C), jnp.float32),
        scratch_shapes=[pltpu.VMEM((G, H), jnp.float32)],
    )(partials, hsums, idx3, stats, bn_gamma[None, :], bn_beta[None, :],
      fc1_w, fc1_b[None, :], fc2_w, fc2_b[None, :])


def kernel(x_in, adj, idx, W_fc, a_w, fc1_w, fc1_b, fc2_w, fc2_b,
           bn_gamma, bn_beta):
    x_p = jnp.pad(x_in, ((0, NP - N), (0, 0)))
    zraw, s_pair, stats = _front(x_p, W_fc, a_w)
    src = adj[0]
    dst = adj[1]
    h_all, hsums = _sc_h(src, dst, s_pair, stats)
    partials = _sc_scatter(src, dst, h_all, zraw)
    idx3 = jnp.pad(idx, (0, NP - N)).reshape(NBLK, 1, BLK)
    return _post(partials, hsums, idx3, stats, bn_gamma, bn_beta, fc1_w,
                 fc1_b, fc2_w, fc2_b)


# speedup vs baseline: 1.4918x; 1.4918x over previous
import functools

import jax
import jax.numpy as jnp
from jax import lax
from jax.experimental import pallas as pl
from jax.experimental.pallas import tpu as pltpu
from jax.experimental.pallas import tpu_sc as plsc

N = 10000
E = 320000
D = 128
H = 128
C = 16
G = 128

NP = 10240
BLK = 1024
NBLK = NP // BLK
NC = 2
NS = 16
NWORK = NC * NS
EPW = E // NWORK
K = 80
NB = EPW // K
NB3 = (NB - 2) // 3
ROWS_PW = NP // NS


def _front_body(x_ref, w_ref, aw_ref, z_ref, sp_ref, stats_ref, ssum, ssq):
    i = pl.program_id(0)

    @pl.when(i == 0)
    def _():
        ssum[0] = 0.0
        ssq[0] = 0.0

    xb = x_ref[...]
    ssum[0] += jnp.sum(xb)
    ssq[0] += jnp.sum(xb * xb)

    w = w_ref[...]
    zb = jnp.dot(xb, w.T, preferred_element_type=jnp.float32)
    z_ref[...] = zb

    aw = aw_ref[...]
    a1 = aw[0, :H]
    a2 = aw[0, H:]
    rid = lax.broadcasted_iota(jnp.int32, (16, H), 0)
    amat = jnp.where(rid == 0, a1[None, :],
                     jnp.where(rid == 1, a2[None, :], 0.0))
    sp_ref[...] = lax.dot_general(amat, zb, (((1,), (1,)), ((), ())),
                                  preferred_element_type=jnp.float32)

    @pl.when(i == pl.num_programs(0) - 1)
    def _():
        cnt = float(N * D)
        mu = ssum[0] / cnt
        var = (ssq[0] - cnt * mu * mu) / (cnt - 1.0)
        rstd = lax.rsqrt(var)
        wsum = jnp.sum(w, axis=1)
        k12 = jnp.sum(wsum * (a1 + a2))
        c12 = mu * rstd * k12
        lane = lax.broadcasted_iota(jnp.int32, (1, H), 1)
        row1 = jnp.where(lane == 0, mu,
                         jnp.where(lane == 1, rstd,
                                   jnp.where(lane == 2, c12, 0.0)))
        stats_ref[0:1, :] = (mu * rstd) * wsum[None, :]
        stats_ref[1:2, :] = row1
        stats_ref[2:8, :] = jnp.zeros((6, H), jnp.float32)


def _front(x_p, W_fc, a_w):
    return pl.pallas_call(
        _front_body,
        grid=(NBLK,),
        in_specs=[
            pl.BlockSpec((BLK, D), lambda i: (i, 0)),
            pl.BlockSpec((H, D), lambda i: (0, 0)),
            pl.BlockSpec((1, 2 * H), lambda i: (0, 0)),
        ],
        out_specs=[
            pl.BlockSpec((BLK, H), lambda i: (i, 0)),
            pl.BlockSpec((16, BLK), lambda i: (0, i)),
            pl.BlockSpec((8, H), lambda i: (0, 0)),
        ],
        out_shape=[
            jax.ShapeDtypeStruct((NP, H), jnp.float32),
            jax.ShapeDtypeStruct((16, NP), jnp.float32),
            jax.ShapeDtypeStruct((8, H), jnp.float32),
        ],
        scratch_shapes=[pltpu.SMEM((1,), jnp.float32),
                        pltpu.SMEM((1,), jnp.float32)],
    )(x_p, W_fc, a_w)


def _sc_h_body(src_hbm, dst_hbm, sp_hbm, stats_hbm, out_h_hbm, out_hs_hbm,
               s1_v, s2_v, st_v, src_v, dst_v, h_v, hsum_v):
    c = lax.axis_index("c")
    s = lax.axis_index("s")
    wid = s * NC + c

    pltpu.sync_copy(sp_hbm.at[0], s1_v)
    pltpu.sync_copy(sp_hbm.at[1], s2_v)
    pltpu.sync_copy(stats_hbm.at[1, pl.ds(0, 16)], st_v)
    stv = st_v[...]
    rstd = stv[1]
    c12 = stv[2]

    def zh(t, carry):
        hsum_v[pl.ds(t * 16, 16)] = jnp.zeros((16,), jnp.float32)
        return carry

    lax.fori_loop(0, NP // 16, zh, 0)

    base0 = wid * EPW
    pltpu.sync_copy(src_hbm.at[pl.ds(base0, EPW)], src_v)
    pltpu.sync_copy(dst_hbm.at[pl.ds(base0, EPW)], dst_v)

    def grp(j, carry):
        s16 = src_v[pl.ds(j * 16, 16)]
        d16 = dst_v[pl.ds(j * 16, 16)]
        g = plsc.load_gather(s1_v, [s16]) + plsc.load_gather(s2_v, [d16])
        lg = g * rstd - c12
        lg = jnp.where(lg >= 0.0, lg, 0.05 * lg)
        h16 = jnp.exp(lg)
        h_v[pl.ds(j * 16, 16)] = h16
        plsc.addupdate_scatter(hsum_v, [s16], h16)
        return carry

    lax.fori_loop(0, EPW // 16, grp, 0)
    pltpu.sync_copy(h_v, out_h_hbm.at[pl.ds(base0, EPW)])
    pltpu.sync_copy(hsum_v, out_hs_hbm.at[wid])


def _sc_h(src, dst, s_pair, stats):
    mesh = plsc.VectorSubcoreMesh(core_axis_name="c", subcore_axis_name="s")
    f = pl.kernel(
        _sc_h_body,
        out_type=[
            jax.ShapeDtypeStruct((E,), jnp.float32),
            jax.ShapeDtypeStruct((NWORK, NP), jnp.float32),
        ],
        mesh=mesh,
        scratch_types=[
            pltpu.VMEM((NP,), jnp.float32),
            pltpu.VMEM((NP,), jnp.float32),
            pltpu.VMEM((16,), jnp.float32),
            pltpu.VMEM((EPW,), jnp.int32),
            pltpu.VMEM((EPW,), jnp.int32),
            pltpu.VMEM((EPW,), jnp.float32),
            pltpu.VMEM((NP,), jnp.float32),
        ],
        compiler_params=pltpu.CompilerParams(needs_layout_passes=False),
    )
    return f(src, dst, s_pair, stats)


NBUF = 4
NB4 = (NB - 1) // NBUF


def _sc_scatter_body(src_hbm, dst_hbm, h_hbm, z_hbm, out_u_hbm,
                     s0_v, s1_v, s2_v, s3_v, d0_v, d1_v, d2_v, d3_v,
                     h0_v, h1_v, h2_v, h3_v, r0_v, r1_v, r2_v, r3_v, u_sh,
                     i0, i1, i2, i3, g0, g1, g2, g3, t0, t1, t2, t3):
    c = lax.axis_index("c")
    s = lax.axis_index("s")
    wid = s * NC + c
    base0 = wid * EPW

    src_bufs = (s0_v, s1_v, s2_v, s3_v)
    dst_bufs = (d0_v, d1_v, d2_v, d3_v)
    h_bufs = (h0_v, h1_v, h2_v, h3_v)
    row_bufs = (r0_v, r1_v, r2_v, r3_v)
    isems = (i0, i1, i2, i3)
    gsems = (g0, g1, g2, g3)
    ssems = (t0, t1, t2, t3)

    def zrow(e, carry):
        for q in range(H // 16):
            r0_v[e, pl.ds(q * 16, 16)] = jnp.zeros((16,), jnp.float32)
        return carry

    lax.fori_loop(0, K, zrow, 0)
    row0 = s * ROWS_PW
    for j in range(ROWS_PW // K):
        pltpu.sync_copy(r0_v, u_sh.at[pl.ds(row0 + j * K, K)])
    plsc.subcore_barrier()

    def stage_idx(b, par):
        pltpu.async_copy(src_hbm.at[pl.ds(base0 + b * K, K)], src_bufs[par],
                         isems[par])
        pltpu.async_copy(dst_hbm.at[pl.ds(base0 + b * K, K)], dst_bufs[par],
                         isems[par])
        pltpu.async_copy(h_hbm.at[pl.ds(base0 + b * K, K)], h_bufs[par],
                         isems[par])

    def wait_idx(par):
        pltpu.make_async_copy(src_hbm.at[pl.ds(base0, K)], src_bufs[par],
                              isems[par]).wait()
        pltpu.make_async_copy(dst_hbm.at[pl.ds(base0, K)], dst_bufs[par],
                              isems[par]).wait()
        pltpu.make_async_copy(h_hbm.at[pl.ds(base0, K)], h_bufs[par],
                              isems[par]).wait()

    def start_gather(par):
        pltpu.async_copy(z_hbm.at[dst_bufs[par]], row_bufs[par], gsems[par])

    def wait_gather(par):
        pltpu.make_async_copy(z_hbm.at[dst_bufs[par]], row_bufs[par],
                              gsems[par]).wait()

    def scale(par):
        rows = row_bufs[par]
        hb = h_bufs[par]

        def erow(e, carry):
            ev = jnp.full((16,), 0, jnp.int32) + e
            hs = plsc.load_gather(hb, [ev])
            for q in range(H // 16):
                rows[e, pl.ds(q * 16, 16)] = rows[e, pl.ds(q * 16, 16)] * hs
            return carry

        lax.fori_loop(0, K, erow, 0)

    def start_scatter(par):
        pltpu.async_copy(row_bufs[par], u_sh.at[src_bufs[par]], ssems[par],
                         add=True)

    def wait_scatter(par):
        pltpu.make_async_copy(row_bufs[par], u_sh.at[src_bufs[par]],
                              ssems[par]).wait()

    stage_idx(0, 0)
    stage_idx(1, 1)
    stage_idx(2, 2)
    wait_idx(0)
    start_gather(0)
    wait_idx(1)
    start_gather(1)

    def step(b, par):
        wait_gather(par)
        scale(par)
        start_scatter(par)
        nb3 = b + 3
        pn3 = (par + 3) % NBUF

        @pl.when(nb3 < NB)
        def _():
            @pl.when(b >= 1)
            def _():
                wait_scatter(pn3)

            stage_idx(nb3, pn3)

        nb2 = b + 2
        pn2 = (par + 2) % NBUF

        @pl.when(nb2 < NB)
        def _():
            wait_idx(pn2)
            start_gather(pn2)

    def quad(q, carry):
        b = q * NBUF
        step(b, 0)
        step(b + 1, 1)
        step(b + 2, 2)
        step(b + 3, 3)
        return carry

    lax.fori_loop(0, NB4, quad, 0)
    step(NB - 1, (NB - 1) % NBUF)

    for b in (NB - 4, NB - 3, NB - 2, NB - 1):
        wait_scatter(b % NBUF)

    plsc.subcore_barrier()
    pltpu.sync_copy(u_sh.at[pl.ds(row0, ROWS_PW)],
                    out_u_hbm.at[c, pl.ds(row0, ROWS_PW)])


def _sc_scatter(src, dst, h_all, zraw):
    mesh = plsc.VectorSubcoreMesh(core_axis_name="c", subcore_axis_name="s")
    f = pl.kernel(
        _sc_scatter_body,
        out_type=jax.ShapeDtypeStruct((NC, NP, H), jnp.float32),
        mesh=mesh,
        scratch_types=(
            [pltpu.VMEM((K,), jnp.int32)] * NBUF
            + [pltpu.VMEM((K,), jnp.int32)] * NBUF
            + [pltpu.VMEM((K,), jnp.float32)] * NBUF
            + [pltpu.VMEM((K, H), jnp.float32)] * NBUF
            + [pltpu.VMEM_SHARED((NP, H), jnp.float32)]
            + [pltpu.SemaphoreType.DMA] * (3 * NBUF)
        ),
        compiler_params=pltpu.CompilerParams(needs_layout_passes=False),
    )
    return f(src, dst, h_all, zraw)


def _post_body(p_ref, h_ref, idx_ref, stats_ref, gamma_ref, beta_ref,
               fc1w_ref, fc1b_ref, fc2w_ref, fc2b_ref, out_ref, acc_ref):
    i = pl.program_id(0)

    @pl.when(i == 0)
    def _():
        acc_ref[...] = jnp.zeros_like(acc_ref)

    u = p_ref[0] + p_ref[1]
    hs = jnp.sum(h_ref[...], axis=0)[:, None]
    muwr = stats_ref[0:1, :]
    rstd = stats_ref[1, 1]
    hs_safe = jnp.where(hs > 0.0, hs, 1.0)
    xh = jnp.maximum((u * rstd - hs * muwr) / hs_safe, 0.0)

    idxb = idx_ref[0, 0, :]
    oh = (idxb[:, None] == lax.broadcasted_iota(jnp.int32, (1, G), 1)
          ).astype(jnp.float32)
    acc_ref[...] += lax.dot_general(oh, xh, (((0,), (0,)), ((), ())),
                                    preferred_element_type=jnp.float32)

    @pl.when(i == pl.num_programs(0) - 1)
    def _():
        pooled = acc_ref[...]
        mean = jnp.mean(pooled, axis=0, keepdims=True)
        var = jnp.mean((pooled - mean) ** 2, axis=0, keepdims=True)
        xb = (pooled - mean) * lax.rsqrt(var + 1e-5)
        xb = xb * gamma_ref[...] + beta_ref[...]
        y = jnp.maximum(
            jnp.dot(xb, fc1w_ref[...].T, preferred_element_type=jnp.float32)
            + fc1b_ref[...], 0.0)
        y = (jnp.dot(y, fc2w_ref[...].T, preferred_element_type=jnp.float32)
             + fc2b_ref[...])
        m = jnp.max(y, axis=1, keepdims=True)
        ly = y - m
        lse = jnp.log(jnp.sum(jnp.exp(ly), axis=1, keepdims=True))
        out_ref[...] = ly - lse


def _post(partials, hsums, idx3, stats, bn_gamma, bn_beta, fc1_w, fc1_b,
          fc2_w, fc2_b):
    return pl.pallas_call(
        _post_body,
        grid=(NBLK,),
        in_specs=[
            pl.BlockSpec((NC, BLK, H), lambda i: (0, i, 0)),
            pl.BlockSpec((NWORK, BLK), lambda i: (0, i)),
            pl.BlockSpec((1, 1, BLK), lambda i: (i, 0, 0)),
            pl.BlockSpec((8, H), lambda i: (0, 0)),
            pl.BlockSpec((1, H), lambda i: (0, 0)),
            pl.BlockSpec((1, H), lambda i: (0, 0)),
            pl.BlockSpec((H, H), lambda i: (0, 0)),
            pl.BlockSpec((1, H), lambda i: (0, 0)),
            pl.BlockSpec((C, H), lambda i: (0, 0)),
            pl.BlockSpec((1, C), lambda i: (0, 0)),
        ],
        out_specs=pl.BlockSpec((G, C), lambda i: (0, 0)),
        out_shape=jax.ShapeDtypeStruct((G, C), jnp.float32),
        scratch_shapes=[pltpu.VMEM((G, H), jnp.float32)],
    )(partials, hsums, idx3, stats, bn_gamma[None, :], bn_beta[None, :],
      fc1_w, fc1_b[None, :], fc2_w, fc2_b[None, :])


def kernel(x_in, adj, idx, W_fc, a_w, fc1_w, fc1_b, fc2_w, fc2_b,
           bn_gamma, bn_beta):
    x_p = jnp.pad(x_in, ((0, NP - N), (0, 0)))
    zraw, s_pair, stats = _front(x_p, W_fc, a_w)
    src = adj[0]
    dst = adj[1]
    h_all, hsums = _sc_h(src, dst, s_pair, stats)
    partials = _sc_scatter(src, dst, h_all, zraw)
    idx3 = jnp.pad(idx, (0, NP - N)).reshape(NBLK, 1, BLK)
    return _post(partials, hsums, idx3, stats, bn_gamma, bn_beta, fc1_w,
                 fc1_b, fc2_w, fc2_b)


# speedup vs baseline: 1.5785x; 1.0581x over previous
import functools

import jax
import jax.numpy as jnp
from jax import lax
from jax.experimental import pallas as pl
from jax.experimental.pallas import tpu as pltpu
from jax.experimental.pallas import tpu_sc as plsc

N = 10000
E = 320000
D = 128
H = 128
C = 16
G = 128

NP = 10240
BLK = 1024
NBLK = NP // BLK
NC = 2
NS = 16
NWORK = NC * NS
EPW = E // NWORK
K = 80
NB = EPW // K
NB3 = (NB - 2) // 3
ROWS_PW = NP // NS


def _front_body(x_ref, w_ref, aw_ref, z_ref, sp_ref, stats_ref, ssum, ssq):
    i = pl.program_id(0)

    @pl.when(i == 0)
    def _():
        ssum[0] = 0.0
        ssq[0] = 0.0

    xb = x_ref[...]
    ssum[0] += jnp.sum(xb)
    ssq[0] += jnp.sum(xb * xb)

    w = w_ref[...]
    zb = jnp.dot(xb, w.T, preferred_element_type=jnp.float32)
    z_ref[...] = zb

    aw = aw_ref[...]
    a1 = aw[0, :H]
    a2 = aw[0, H:]
    rid = lax.broadcasted_iota(jnp.int32, (16, H), 0)
    amat = jnp.where(rid == 0, a1[None, :],
                     jnp.where(rid == 1, a2[None, :], 0.0))
    sp_ref[...] = lax.dot_general(amat, zb, (((1,), (1,)), ((), ())),
                                  preferred_element_type=jnp.float32)

    @pl.when(i == pl.num_programs(0) - 1)
    def _():
        cnt = float(N * D)
        mu = ssum[0] / cnt
        var = (ssq[0] - cnt * mu * mu) / (cnt - 1.0)
        rstd = lax.rsqrt(var)
        wsum = jnp.sum(w, axis=1)
        k12 = jnp.sum(wsum * (a1 + a2))
        c12 = mu * rstd * k12
        lane = lax.broadcasted_iota(jnp.int32, (1, H), 1)
        row1 = jnp.where(lane == 0, mu,
                         jnp.where(lane == 1, rstd,
                                   jnp.where(lane == 2, c12, 0.0)))
        stats_ref[0:1, :] = (mu * rstd) * wsum[None, :]
        stats_ref[1:2, :] = row1
        stats_ref[2:8, :] = jnp.zeros((6, H), jnp.float32)


def _front(x_p, W_fc, a_w):
    return pl.pallas_call(
        _front_body,
        grid=(NBLK,),
        in_specs=[
            pl.BlockSpec((BLK, D), lambda i: (i, 0)),
            pl.BlockSpec((H, D), lambda i: (0, 0)),
            pl.BlockSpec((1, 2 * H), lambda i: (0, 0)),
        ],
        out_specs=[
            pl.BlockSpec((BLK, H), lambda i: (i, 0)),
            pl.BlockSpec((16, BLK), lambda i: (0, i)),
            pl.BlockSpec((8, H), lambda i: (0, 0)),
        ],
        out_shape=[
            jax.ShapeDtypeStruct((NP, H), jnp.float32),
            jax.ShapeDtypeStruct((16, NP), jnp.float32),
            jax.ShapeDtypeStruct((8, H), jnp.float32),
        ],
        scratch_shapes=[pltpu.SMEM((1,), jnp.float32),
                        pltpu.SMEM((1,), jnp.float32)],
    )(x_p, W_fc, a_w)


def _sc_h_body(adj_hbm, sp_hbm, stats_hbm, out_h_hbm, out_hs_hbm,
               s1_v, s2_v, st_v, src_v, dst_v, h_v, hsum_v):
    c = lax.axis_index("c")
    s = lax.axis_index("s")
    wid = s * NC + c

    pltpu.sync_copy(sp_hbm.at[0], s1_v)
    pltpu.sync_copy(sp_hbm.at[1], s2_v)
    pltpu.sync_copy(stats_hbm.at[1, pl.ds(0, 16)], st_v)
    stv = st_v[...]
    rstd = stv[1]
    c12 = stv[2]

    def zh(t, carry):
        hsum_v[pl.ds(t * 16, 16)] = jnp.zeros((16,), jnp.float32)
        return carry

    lax.fori_loop(0, NP // 16, zh, 0)

    base0 = wid * EPW
    pltpu.sync_copy(adj_hbm.at[pl.ds(base0, EPW)], src_v)
    pltpu.sync_copy(adj_hbm.at[pl.ds(E + base0, EPW)], dst_v)

    def one(j):
        s16 = src_v[pl.ds(j * 16, 16)]
        d16 = dst_v[pl.ds(j * 16, 16)]
        g = plsc.load_gather(s1_v, [s16]) + plsc.load_gather(s2_v, [d16])
        lg = g * rstd - c12
        lg = jnp.where(lg >= 0.0, lg, 0.05 * lg)
        h16 = jnp.exp(lg)
        h_v[pl.ds(j * 16, 16)] = h16
        plsc.addupdate_scatter(hsum_v, [s16], h16)

    UN = 5

    def grp(j4, carry):
        for t in range(UN):
            one(j4 * UN + t)
        return carry

    lax.fori_loop(0, EPW // 16 // UN, grp, 0)
    pltpu.sync_copy(h_v, out_h_hbm.at[pl.ds(base0, EPW)])
    pltpu.sync_copy(hsum_v, out_hs_hbm.at[wid])


def _sc_h(adj, s_pair, stats):
    mesh = plsc.VectorSubcoreMesh(core_axis_name="c", subcore_axis_name="s")
    f = pl.kernel(
        _sc_h_body,
        out_type=[
            jax.ShapeDtypeStruct((E,), jnp.float32),
            jax.ShapeDtypeStruct((NWORK, NP), jnp.float32),
        ],
        mesh=mesh,
        scratch_types=[
            pltpu.VMEM((NP,), jnp.float32),
            pltpu.VMEM((NP,), jnp.float32),
            pltpu.VMEM((16,), jnp.float32),
            pltpu.VMEM((EPW,), jnp.int32),
            pltpu.VMEM((EPW,), jnp.int32),
            pltpu.VMEM((EPW,), jnp.float32),
            pltpu.VMEM((NP,), jnp.float32),
        ],
        compiler_params=pltpu.CompilerParams(needs_layout_passes=False),
    )
    return f(adj, s_pair, stats)


NBUF = 4
NB4 = (NB - 1) // NBUF


def _sc_scatter_body(adj_hbm, h_hbm, z_hbm, out_u_hbm,
                     s0_v, s1_v, s2_v, s3_v, d0_v, d1_v, d2_v, d3_v,
                     h0_v, h1_v, h2_v, h3_v, r0_v, r1_v, r2_v, r3_v, u_sh,
                     i0, i1, i2, i3, g0, g1, g2, g3, t0, t1, t2, t3):
    c = lax.axis_index("c")
    s = lax.axis_index("s")
    wid = s * NC + c
    base0 = wid * EPW

    src_bufs = (s0_v, s1_v, s2_v, s3_v)
    dst_bufs = (d0_v, d1_v, d2_v, d3_v)
    h_bufs = (h0_v, h1_v, h2_v, h3_v)
    row_bufs = (r0_v, r1_v, r2_v, r3_v)
    isems = (i0, i1, i2, i3)
    gsems = (g0, g1, g2, g3)
    ssems = (t0, t1, t2, t3)

    def zrow(e, carry):
        for q in range(H // 16):
            r0_v[e, pl.ds(q * 16, 16)] = jnp.zeros((16,), jnp.float32)
        return carry

    lax.fori_loop(0, K, zrow, 0)
    row0 = s * ROWS_PW
    for j in range(ROWS_PW // K):
        pltpu.sync_copy(r0_v, u_sh.at[pl.ds(row0 + j * K, K)])
    plsc.subcore_barrier()

    def stage_idx(b, par):
        pltpu.async_copy(adj_hbm.at[pl.ds(base0 + b * K, K)],
                         src_bufs[par], isems[par])
        pltpu.async_copy(adj_hbm.at[pl.ds(E + base0 + b * K, K)],
                         dst_bufs[par], isems[par])
        pltpu.async_copy(h_hbm.at[pl.ds(base0 + b * K, K)], h_bufs[par],
                         isems[par])

    def wait_idx(par):
        pltpu.make_async_copy(adj_hbm.at[pl.ds(base0, K)], src_bufs[par],
                              isems[par]).wait()
        pltpu.make_async_copy(adj_hbm.at[pl.ds(E + base0, K)], dst_bufs[par],
                              isems[par]).wait()
        pltpu.make_async_copy(h_hbm.at[pl.ds(base0, K)], h_bufs[par],
                              isems[par]).wait()

    def start_gather(par):
        pltpu.async_copy(z_hbm.at[dst_bufs[par]], row_bufs[par], gsems[par])

    def wait_gather(par):
        pltpu.make_async_copy(z_hbm.at[dst_bufs[par]], row_bufs[par],
                              gsems[par]).wait()

    def scale(par):
        rows = row_bufs[par]
        hb = h_bufs[par]

        def erow(e2, carry):
            for t in range(2):
                e = e2 * 2 + t
                ev = jnp.full((16,), 0, jnp.int32) + e
                hs = plsc.load_gather(hb, [ev])
                for q in range(H // 16):
                    rows[e, pl.ds(q * 16, 16)] = (
                        rows[e, pl.ds(q * 16, 16)] * hs)
            return carry

        lax.fori_loop(0, K // 2, erow, 0)

    def start_scatter(par):
        pltpu.async_copy(row_bufs[par], u_sh.at[src_bufs[par]], ssems[par],
                         add=True)

    def wait_scatter(par):
        pltpu.make_async_copy(row_bufs[par], u_sh.at[src_bufs[par]],
                              ssems[par]).wait()

    stage_idx(0, 0)
    stage_idx(1, 1)
    stage_idx(2, 2)
    wait_idx(0)
    start_gather(0)
    wait_idx(1)
    start_gather(1)

    def step(b, par):
        wait_gather(par)
        scale(par)
        start_scatter(par)
        nb3 = b + 3
        pn3 = (par + 3) % NBUF

        @pl.when(nb3 < NB)
        def _():
            @pl.when(b >= 1)
            def _():
                wait_scatter(pn3)

            stage_idx(nb3, pn3)

        nb2 = b + 2
        pn2 = (par + 2) % NBUF

        @pl.when(nb2 < NB)
        def _():
            wait_idx(pn2)
            start_gather(pn2)

    def quad(q, carry):
        b = q * NBUF
        step(b, 0)
        step(b + 1, 1)
        step(b + 2, 2)
        step(b + 3, 3)
        return carry

    lax.fori_loop(0, NB4, quad, 0)
    step(NB - 1, (NB - 1) % NBUF)

    for b in (NB - 4, NB - 3, NB - 2, NB - 1):
        wait_scatter(b % NBUF)

    plsc.subcore_barrier()
    pltpu.sync_copy(u_sh.at[pl.ds(row0, ROWS_PW)],
                    out_u_hbm.at[c, pl.ds(row0, ROWS_PW)])


def _sc_scatter(adj, h_all, zraw):
    mesh = plsc.VectorSubcoreMesh(core_axis_name="c", subcore_axis_name="s")
    f = pl.kernel(
        _sc_scatter_body,
        out_type=jax.ShapeDtypeStruct((NC, NP, H), jnp.float32),
        mesh=mesh,
        scratch_types=(
            [pltpu.VMEM((K,), jnp.int32)] * NBUF
            + [pltpu.VMEM((K,), jnp.int32)] * NBUF
            + [pltpu.VMEM((K,), jnp.float32)] * NBUF
            + [pltpu.VMEM((K, H), jnp.float32)] * NBUF
            + [pltpu.VMEM_SHARED((NP, H), jnp.float32)]
            + [pltpu.SemaphoreType.DMA] * (3 * NBUF)
        ),
        compiler_params=pltpu.CompilerParams(needs_layout_passes=False),
    )
    return f(adj, h_all, zraw)


def _post_body(p_ref, h_ref, idx_ref, stats_ref, gamma_ref, beta_ref,
               fc1w_ref, fc1b_ref, fc2w_ref, fc2b_ref, out_ref, acc_ref):
    i = pl.program_id(0)

    @pl.when(i == 0)
    def _():
        acc_ref[...] = jnp.zeros_like(acc_ref)

    u = p_ref[0] + p_ref[1]
    hs = jnp.sum(h_ref[...], axis=0)[:, None]
    muwr = stats_ref[0:1, :]
    rstd = stats_ref[1, 1]
    hs_safe = jnp.where(hs > 0.0, hs, 1.0)
    xh = jnp.maximum((u * rstd - hs * muwr) / hs_safe, 0.0)

    idxb = idx_ref[0, 0, :]
    oh = (idxb[:, None] == lax.broadcasted_iota(jnp.int32, (1, G), 1)
          ).astype(jnp.float32)
    acc_ref[...] += lax.dot_general(oh, xh, (((0,), (0,)), ((), ())),
                                    preferred_element_type=jnp.float32)

    @pl.when(i == pl.num_programs(0) - 1)
    def _():
        pooled = acc_ref[...]
        mean = jnp.mean(pooled, axis=0, keepdims=True)
        var = jnp.mean((pooled - mean) ** 2, axis=0, keepdims=True)
        xb = (pooled - mean) * lax.rsqrt(var + 1e-5)
        xb = xb * gamma_ref[...] + beta_ref[...]
        y = jnp.maximum(
            jnp.dot(xb, fc1w_ref[...].T, preferred_element_type=jnp.float32)
            + fc1b_ref[...], 0.0)
        y = (jnp.dot(y, fc2w_ref[...].T, preferred_element_type=jnp.float32)
             + fc2b_ref[...])
        m = jnp.max(y, axis=1, keepdims=True)
        ly = y - m
        lse = jnp.log(jnp.sum(jnp.exp(ly), axis=1, keepdims=True))
        out_ref[...] = ly - lse


def _post(partials, hsums, idx3, stats, bn_gamma, bn_beta, fc1_w, fc1_b,
          fc2_w, fc2_b):
    return pl.pallas_call(
        _post_body,
        grid=(NBLK,),
        in_specs=[
            pl.BlockSpec((NC, BLK, H), lambda i: (0, i, 0)),
            pl.BlockSpec((NWORK, BLK), lambda i: (0, i)),
            pl.BlockSpec((1, 1, BLK), lambda i: (i, 0, 0)),
            pl.BlockSpec((8, H), lambda i: (0, 0)),
            pl.BlockSpec((1, H), lambda i: (0, 0)),
            pl.BlockSpec((1, H), lambda i: (0, 0)),
            pl.BlockSpec((H, H), lambda i: (0, 0)),
            pl.BlockSpec((1, H), lambda i: (0, 0)),
            pl.BlockSpec((C, H), lambda i: (0, 0)),
            pl.BlockSpec((1, C), lambda i: (0, 0)),
        ],
        out_specs=pl.BlockSpec((G, C), lambda i: (0, 0)),
        out_shape=jax.ShapeDtypeStruct((G, C), jnp.float32),
        scratch_shapes=[pltpu.VMEM((G, H), jnp.float32)],
    )(partials, hsums, idx3, stats, bn_gamma[None, :], bn_beta[None, :],
      fc1_w, fc1_b[None, :], fc2_w, fc2_b[None, :])


def kernel(x_in, adj, idx, W_fc, a_w, fc1_w, fc1_b, fc2_w, fc2_b,
           bn_gamma, bn_beta):
    x_p = jnp.pad(x_in, ((0, NP - N), (0, 0)))
    zraw, s_pair, stats = _front(x_p, W_fc, a_w)
    adj_flat = jnp.reshape(adj, (2 * E,))
    h_all, hsums = _sc_h(adj_flat, s_pair, stats)
    partials = _sc_scatter(adj_flat, h_all, zraw)
    idx3 = jnp.pad(idx, (0, NP - N)).reshape(NBLK, 1, BLK)
    return _post(partials, hsums, idx3, stats, bn_gamma, bn_beta, fc1_w,
                 fc1_b, fc2_w, fc2_b)


# speedup vs baseline: 1.8578x; 1.1769x over previous
import functools

import jax
import jax.numpy as jnp
from jax import lax
from jax.experimental import pallas as pl
from jax.experimental.pallas import tpu as pltpu
from jax.experimental.pallas import tpu_sc as plsc

N = 10000
E = 320000
D = 128
H = 128
C = 16
G = 128

NP = 10240
BLK = 2048
NBLK = NP // BLK
NC = 2
NS = 16
NWORK = NC * NS
EPW = E // NWORK
K = 80
NB = EPW // K
NB3 = (NB - 2) // 3
ROWS_PW = NP // NS


def _front_body(x_ref, w_ref, aw_ref, z_ref, sp_ref, stats_ref, ssum, ssq):
    i = pl.program_id(0)

    @pl.when(i == 0)
    def _():
        ssum[0] = 0.0
        ssq[0] = 0.0

    xb = x_ref[...]
    ssum[0] += jnp.sum(xb)
    ssq[0] += jnp.sum(xb * xb)

    w = w_ref[...]
    zb = jnp.dot(xb, w.T, preferred_element_type=jnp.float32)
    z_ref[...] = zb

    aw = aw_ref[...]
    a1 = aw[0, :H]
    a2 = aw[0, H:]
    rid = lax.broadcasted_iota(jnp.int32, (16, H), 0)
    amat = jnp.where(rid == 0, a1[None, :],
                     jnp.where(rid == 1, a2[None, :], 0.0))
    sp_ref[...] = lax.dot_general(amat, zb, (((1,), (1,)), ((), ())),
                                  preferred_element_type=jnp.float32)

    @pl.when(i == pl.num_programs(0) - 1)
    def _():
        cnt = float(N * D)
        mu = ssum[0] / cnt
        var = (ssq[0] - cnt * mu * mu) / (cnt - 1.0)
        rstd = lax.rsqrt(var)
        wsum = jnp.sum(w, axis=1)
        k12 = jnp.sum(wsum * (a1 + a2))
        c12 = mu * rstd * k12
        lane = lax.broadcasted_iota(jnp.int32, (1, H), 1)
        row1 = jnp.where(lane == 0, mu,
                         jnp.where(lane == 1, rstd,
                                   jnp.where(lane == 2, c12, 0.0)))
        stats_ref[0:1, :] = (mu * rstd) * wsum[None, :]
        stats_ref[1:2, :] = row1
        stats_ref[2:8, :] = jnp.zeros((6, H), jnp.float32)


def _front(x_p, W_fc, a_w):
    return pl.pallas_call(
        _front_body,
        grid=(NBLK,),
        in_specs=[
            pl.BlockSpec((BLK, D), lambda i: (i, 0)),
            pl.BlockSpec((H, D), lambda i: (0, 0)),
            pl.BlockSpec((1, 2 * H), lambda i: (0, 0)),
        ],
        out_specs=[
            pl.BlockSpec((BLK, H), lambda i: (i, 0)),
            pl.BlockSpec((16, BLK), lambda i: (0, i)),
            pl.BlockSpec((8, H), lambda i: (0, 0)),
        ],
        out_shape=[
            jax.ShapeDtypeStruct((NP, H), jnp.float32),
            jax.ShapeDtypeStruct((16, NP), jnp.float32),
            jax.ShapeDtypeStruct((8, H), jnp.float32),
        ],
        scratch_shapes=[pltpu.SMEM((1,), jnp.float32),
                        pltpu.SMEM((1,), jnp.float32)],
    )(x_p, W_fc, a_w)


def _sc_h_body(adj_hbm, sp_hbm, stats_hbm, out_h_hbm, out_hs_hbm,
               s1_v, s2_v, st_v, src_v, dst_v, h_v, hsum_v):
    c = lax.axis_index("c")
    s = lax.axis_index("s")
    wid = s * NC + c

    pltpu.sync_copy(sp_hbm.at[0], s1_v)
    pltpu.sync_copy(sp_hbm.at[1], s2_v)
    pltpu.sync_copy(stats_hbm.at[1, pl.ds(0, 16)], st_v)
    stv = st_v[...]
    rstd = stv[1]
    c12 = stv[2]

    def zh(t, carry):
        hsum_v[pl.ds(t * 16, 16)] = jnp.zeros((16,), jnp.float32)
        return carry

    lax.fori_loop(0, NP // 16, zh, 0)

    base0 = wid * EPW
    pltpu.sync_copy(adj_hbm.at[pl.ds(base0, EPW)], src_v)
    pltpu.sync_copy(adj_hbm.at[pl.ds(E + base0, EPW)], dst_v)

    UN = 5

    def grp(j4, carry):
        acc = []
        for t in range(UN):
            j = j4 * UN + t
            s16 = src_v[pl.ds(j * 16, 16)]
            d16 = dst_v[pl.ds(j * 16, 16)]
            g = plsc.load_gather(s1_v, [s16]) + plsc.load_gather(s2_v, [d16])
            lg = g * rstd - c12
            lg = jnp.where(lg >= 0.0, lg, 0.05 * lg)
            acc.append((s16, jnp.exp(lg)))
        for t, (s16, h16) in enumerate(acc):
            h_v[pl.ds((j4 * UN + t) * 16, 16)] = h16
        for t, (s16, h16) in enumerate(acc):
            plsc.addupdate_scatter(hsum_v, [s16], h16)
        return carry

    lax.fori_loop(0, EPW // 16 // UN, grp, 0)
    pltpu.sync_copy(h_v, out_h_hbm.at[pl.ds(base0, EPW)])
    pltpu.sync_copy(hsum_v, out_hs_hbm.at[wid])


def _sc_h(adj, s_pair, stats):
    mesh = plsc.VectorSubcoreMesh(core_axis_name="c", subcore_axis_name="s")
    f = pl.kernel(
        _sc_h_body,
        out_type=[
            jax.ShapeDtypeStruct((E,), jnp.float32),
            jax.ShapeDtypeStruct((NWORK, NP), jnp.float32),
        ],
        mesh=mesh,
        scratch_types=[
            pltpu.VMEM((NP,), jnp.float32),
            pltpu.VMEM((NP,), jnp.float32),
            pltpu.VMEM((16,), jnp.float32),
            pltpu.VMEM((EPW,), jnp.int32),
            pltpu.VMEM((EPW,), jnp.int32),
            pltpu.VMEM((EPW,), jnp.float32),
            pltpu.VMEM((NP,), jnp.float32),
        ],
        compiler_params=pltpu.CompilerParams(needs_layout_passes=False),
    )
    return f(adj, s_pair, stats)


NBUF = 4
NB4 = (NB - 1) // NBUF


def _sc_scatter_body(adj_hbm, h_hbm, z_hbm, out_u_hbm,
                     s0_v, s1_v, s2_v, s3_v, d0_v, d1_v, d2_v, d3_v,
                     h0_v, h1_v, h2_v, h3_v, r0_v, r1_v, r2_v, r3_v, u_sh,
                     i0, i1, i2, i3, g0, g1, g2, g3, t0, t1, t2, t3):
    c = lax.axis_index("c")
    s = lax.axis_index("s")
    wid = s * NC + c
    base0 = wid * EPW

    src_bufs = (s0_v, s1_v, s2_v, s3_v)
    dst_bufs = (d0_v, d1_v, d2_v, d3_v)
    h_bufs = (h0_v, h1_v, h2_v, h3_v)
    row_bufs = (r0_v, r1_v, r2_v, r3_v)
    isems = (i0, i1, i2, i3)
    gsems = (g0, g1, g2, g3)
    ssems = (t0, t1, t2, t3)

    def zrow(e, carry):
        for q in range(H // 16):
            r0_v[e, pl.ds(q * 16, 16)] = jnp.zeros((16,), jnp.float32)
        return carry

    lax.fori_loop(0, K, zrow, 0)
    row0 = s * ROWS_PW
    for j in range(ROWS_PW // K):
        pltpu.sync_copy(r0_v, u_sh.at[pl.ds(row0 + j * K, K)])
    plsc.subcore_barrier()

    def stage_idx(b, par):
        pltpu.async_copy(adj_hbm.at[pl.ds(base0 + b * K, K)],
                         src_bufs[par], isems[par])
        pltpu.async_copy(adj_hbm.at[pl.ds(E + base0 + b * K, K)],
                         dst_bufs[par], isems[par])
        pltpu.async_copy(h_hbm.at[pl.ds(base0 + b * K, K)], h_bufs[par],
                         isems[par])

    def wait_idx(par):
        pltpu.make_async_copy(adj_hbm.at[pl.ds(base0, K)], src_bufs[par],
                              isems[par]).wait()
        pltpu.make_async_copy(adj_hbm.at[pl.ds(E + base0, K)], dst_bufs[par],
                              isems[par]).wait()
        pltpu.make_async_copy(h_hbm.at[pl.ds(base0, K)], h_bufs[par],
                              isems[par]).wait()

    def start_gather(par):
        pltpu.async_copy(z_hbm.at[dst_bufs[par]], row_bufs[par], gsems[par])

    def wait_gather(par):
        pltpu.make_async_copy(z_hbm.at[dst_bufs[par]], row_bufs[par],
                              gsems[par]).wait()

    def scale(par):
        rows = row_bufs[par]
        hb = h_bufs[par]

        def erow(e4, carry):
            hss = []
            for t in range(4):
                e = e4 * 4 + t
                ev = jnp.full((16,), 0, jnp.int32) + e
                hss.append(plsc.load_gather(hb, [ev]))
            for t in range(4):
                e = e4 * 4 + t
                for q in range(H // 16):
                    rows[e, pl.ds(q * 16, 16)] = (
                        rows[e, pl.ds(q * 16, 16)] * hss[t])
            return carry

        lax.fori_loop(0, K // 4, erow, 0)

    def start_scatter(par):
        pltpu.async_copy(row_bufs[par], u_sh.at[src_bufs[par]], ssems[par],
                         add=True)

    def wait_scatter(par):
        pltpu.make_async_copy(row_bufs[par], u_sh.at[src_bufs[par]],
                              ssems[par]).wait()

    stage_idx(0, 0)
    stage_idx(1, 1)
    stage_idx(2, 2)
    wait_idx(0)
    start_gather(0)
    wait_idx(1)
    start_gather(1)

    def step(b, par):
        wait_gather(par)
        scale(par)
        start_scatter(par)
        nb3 = b + 3
        pn3 = (par + 3) % NBUF

        @pl.when(nb3 < NB)
        def _():
            @pl.when(b >= 1)
            def _():
                wait_scatter(pn3)

            stage_idx(nb3, pn3)

        nb2 = b + 2
        pn2 = (par + 2) % NBUF

        @pl.when(nb2 < NB)
        def _():
            wait_idx(pn2)
            start_gather(pn2)

    def quad(q, carry):
        b = q * NBUF
        step(b, 0)
        step(b + 1, 1)
        step(b + 2, 2)
        step(b + 3, 3)
        return carry

    lax.fori_loop(0, NB4, quad, 0)
    step(NB - 1, (NB - 1) % NBUF)

    for b in (NB - 4, NB - 3, NB - 2, NB - 1):
        wait_scatter(b % NBUF)

    plsc.subcore_barrier()
    pltpu.sync_copy(u_sh.at[pl.ds(row0, ROWS_PW)],
                    out_u_hbm.at[c, pl.ds(row0, ROWS_PW)])


def _sc_scatter(adj, h_all, zraw):
    mesh = plsc.VectorSubcoreMesh(core_axis_name="c", subcore_axis_name="s")
    f = pl.kernel(
        _sc_scatter_body,
        out_type=jax.ShapeDtypeStruct((NC, NP, H), jnp.float32),
        mesh=mesh,
        scratch_types=(
            [pltpu.VMEM((K,), jnp.int32)] * NBUF
            + [pltpu.VMEM((K,), jnp.int32)] * NBUF
            + [pltpu.VMEM((K,), jnp.float32)] * NBUF
            + [pltpu.VMEM((K, H), jnp.float32)] * NBUF
            + [pltpu.VMEM_SHARED((NP, H), jnp.float32)]
            + [pltpu.SemaphoreType.DMA] * (3 * NBUF)
        ),
        compiler_params=pltpu.CompilerParams(needs_layout_passes=False),
    )
    return f(adj, h_all, zraw)


def _post_body(p_ref, h_ref, idx_ref, stats_ref, gamma_ref, beta_ref,
               fc1w_ref, fc1b_ref, fc2w_ref, fc2b_ref, out_ref, acc_ref):
    i = pl.program_id(0)

    @pl.when(i == 0)
    def _():
        acc_ref[...] = jnp.zeros_like(acc_ref)

    u = p_ref[0] + p_ref[1]
    hs = jnp.sum(h_ref[...], axis=0)[:, None]
    muwr = stats_ref[0:1, :]
    rstd = stats_ref[1, 1]
    hs_safe = jnp.where(hs > 0.0, hs, 1.0)
    xh = jnp.maximum((u * rstd - hs * muwr) / hs_safe, 0.0)

    idxb = idx_ref[0, 0, :]
    oh = (idxb[:, None] == lax.broadcasted_iota(jnp.int32, (1, G), 1)
          ).astype(jnp.float32)
    acc_ref[...] += lax.dot_general(oh, xh, (((0,), (0,)), ((), ())),
                                    preferred_element_type=jnp.float32)

    @pl.when(i == pl.num_programs(0) - 1)
    def _():
        pooled = acc_ref[...]
        mean = jnp.mean(pooled, axis=0, keepdims=True)
        var = jnp.mean((pooled - mean) ** 2, axis=0, keepdims=True)
        xb = (pooled - mean) * lax.rsqrt(var + 1e-5)
        xb = xb * gamma_ref[...] + beta_ref[...]
        y = jnp.maximum(
            jnp.dot(xb, fc1w_ref[...].T, preferred_element_type=jnp.float32)
            + fc1b_ref[...], 0.0)
        y = (jnp.dot(y, fc2w_ref[...].T, preferred_element_type=jnp.float32)
             + fc2b_ref[...])
        m = jnp.max(y, axis=1, keepdims=True)
        ly = y - m
        lse = jnp.log(jnp.sum(jnp.exp(ly), axis=1, keepdims=True))
        out_ref[...] = ly - lse


def _post(partials, hsums, idx3, stats, bn_gamma, bn_beta, fc1_w, fc1_b,
          fc2_w, fc2_b):
    return pl.pallas_call(
        _post_body,
        grid=(NBLK,),
        in_specs=[
            pl.BlockSpec((NC, BLK, H), lambda i: (0, i, 0)),
            pl.BlockSpec((NWORK, BLK), lambda i: (0, i)),
            pl.BlockSpec((1, 1, BLK), lambda i: (i, 0, 0)),
            pl.BlockSpec((8, H), lambda i: (0, 0)),
            pl.BlockSpec((1, H), lambda i: (0, 0)),
            pl.BlockSpec((1, H), lambda i: (0, 0)),
            pl.BlockSpec((H, H), lambda i: (0, 0)),
            pl.BlockSpec((1, H), lambda i: (0, 0)),
            pl.BlockSpec((C, H), lambda i: (0, 0)),
            pl.BlockSpec((1, C), lambda i: (0, 0)),
        ],
        out_specs=pl.BlockSpec((G, C), lambda i: (0, 0)),
        out_shape=jax.ShapeDtypeStruct((G, C), jnp.float32),
        scratch_shapes=[pltpu.VMEM((G, H), jnp.float32)],
    )(partials, hsums, idx3, stats, bn_gamma[None, :], bn_beta[None, :],
      fc1_w, fc1_b[None, :], fc2_w, fc2_b[None, :])


def kernel(x_in, adj, idx, W_fc, a_w, fc1_w, fc1_b, fc2_w, fc2_b,
           bn_gamma, bn_beta):
    x_p = jnp.pad(x_in, ((0, NP - N), (0, 0)))
    zraw, s_pair, stats = _front(x_p, W_fc, a_w)
    adj_flat = jnp.reshape(adj, (2 * E,))
    h_all, hsums = _sc_h(adj_flat, s_pair, stats)
    partials = _sc_scatter(adj_flat, h_all, zraw)
    idx3 = jnp.pad(idx, (0, NP - N)).reshape(NBLK, 1, BLK)
    return _post(partials, hsums, idx3, stats, bn_gamma, bn_beta, fc1_w,
                 fc1_b, fc2_w, fc2_b)
